# Initial kernel scaffold; baseline (speedup 1.0000x reference)
import jax
import jax.numpy as jnp
from jax.experimental import pallas as pl


def kernel(x, labels):
    raise NotImplementedError("write your pallas kernel here")


# speedup vs baseline: 8.2350x; 8.2350x over previous
import functools

import jax
import jax.numpy as jnp
from jax.experimental import pallas as pl
from jax.experimental.pallas import tpu as pltpu

N = 1000
C = 81
M = 28
D = M * M
BM = 8


def _body(lab_ref, x_ref, o_ref):
    lab = lab_ref[...]
    acc = x_ref[:, 0, :]
    for c in range(1, C):
        acc = jnp.where(lab == c, x_ref[:, c, :], acc)
    o_ref[...] = jax.nn.sigmoid(acc)


@jax.jit
def _select_sigmoid(lab2, xt):
    return pl.pallas_call(
        _body,
        grid=(D // BM,),
        in_specs=[
            pl.BlockSpec((1, N), lambda i: (0, 0)),
            pl.BlockSpec((BM, C, N), lambda i: (i, 0, 0)),
        ],
        out_specs=pl.BlockSpec((BM, N), lambda i: (i, 0)),
        out_shape=jax.ShapeDtypeStruct((D, N), jnp.float32),
    )(lab2, xt)


def kernel(x, labels):
    xt = jnp.transpose(x, (2, 3, 1, 0)).reshape(D, C, N)
    lab2 = labels.astype(jnp.int32).reshape(1, N)
    out_t = _select_sigmoid(lab2, xt)
    return out_t.T.reshape(N, 1, M, M)


# speedup vs baseline: 13.1345x; 1.5950x over previous
import functools

import jax
import jax.numpy as jnp
from jax.experimental import pallas as pl
from jax.experimental.pallas import tpu as pltpu

N = 1000
C = 81
M = 28
D = M * M
BM = 16
CT = C // 8


def _body(lab_ref, x_ref, o_ref):
    lab = lab_ref[...]
    labb = jnp.broadcast_to(lab, (8, N))
    sub = jax.lax.broadcasted_iota(jnp.int32, (8, N), 0)
    masks = [labb == (8 * t + sub) for t in range(CT)]
    last = lab == (C - 1)
    zero8 = jnp.zeros((8, N), jnp.float32)
    zero1 = jnp.zeros((1, N), jnp.float32)
    for m in range(BM):
        acc = zero8
        for t in range(CT):
            acc = acc + jnp.where(masks[t], x_ref[m, 8 * t : 8 * t + 8, :], zero8)
        row = jnp.sum(acc, axis=0, keepdims=True)
        row = row + jnp.where(last, x_ref[m, C - 1 : C, :], zero1)
        o_ref[pl.ds(m, 1), :] = jax.nn.sigmoid(row)


@jax.jit
def _select_sigmoid(lab2, xt):
    return pl.pallas_call(
        _body,
        grid=(D // BM,),
        in_specs=[
            pl.BlockSpec((1, N), lambda i: (0, 0)),
            pl.BlockSpec((BM, C, N), lambda i: (i, 0, 0)),
        ],
        out_specs=pl.BlockSpec((BM, N), lambda i: (i, 0)),
        out_shape=jax.ShapeDtypeStruct((D, N), jnp.float32),
    )(lab2, xt)


def kernel(x, labels):
    xt = jnp.transpose(x, (2, 3, 1, 0)).reshape(D, C, N)
    lab2 = labels.astype(jnp.int32).reshape(1, N)
    out_t = _select_sigmoid(lab2, xt)
    return out_t.T.reshape(N, 1, M, M)


# speedup vs baseline: 13.5173x; 1.0291x over previous
import functools

import jax
import jax.numpy as jnp
from jax.experimental import pallas as pl
from jax.experimental.pallas import tpu as pltpu

N = 1000
C = 81
M = 28
D = M * M
BM = 56
CT = C // 8


def _body(lab_ref, x_ref, o_ref):
    lab = lab_ref[...]
    labb = jnp.broadcast_to(lab, (8, N))
    sub = jax.lax.broadcasted_iota(jnp.int32, (8, N), 0)
    masks = [labb == (8 * t + sub) for t in range(CT)]
    last = lab == (C - 1)
    zero8 = jnp.zeros((8, N), jnp.float32)
    zero1 = jnp.zeros((1, N), jnp.float32)
    for m in range(BM):
        acc = zero8
        for t in range(CT):
            acc = acc + jnp.where(masks[t], x_ref[m, 8 * t : 8 * t + 8, :], zero8)
        row = jnp.sum(acc, axis=0, keepdims=True)
        row = row + jnp.where(last, x_ref[m, C - 1 : C, :], zero1)
        o_ref[pl.ds(m, 1), :] = jax.nn.sigmoid(row)


@jax.jit
def _select_sigmoid(lab2, xt):
    return pl.pallas_call(
        _body,
        grid=(D // BM,),
        in_specs=[
            pl.BlockSpec((1, N), lambda i: (0, 0)),
            pl.BlockSpec((BM, C, N), lambda i: (i, 0, 0)),
        ],
        out_specs=pl.BlockSpec((BM, N), lambda i: (i, 0)),
        out_shape=jax.ShapeDtypeStruct((D, N), jnp.float32),
    )(lab2, xt)


def kernel(x, labels):
    xt = jnp.transpose(x, (2, 3, 1, 0)).reshape(D, C, N)
    lab2 = labels.astype(jnp.int32).reshape(1, N)
    out_t = _select_sigmoid(lab2, xt)
    return out_t.T.reshape(N, 1, M, M)
